# baseline (speedup 1.0000x reference)
import functools

import jax
import jax.numpy as jnp
from jax import lax
from jax.experimental import pallas as pl
from jax.experimental.pallas import tpu as pltpu
from jax.experimental.pallas import tpu_sc as plsc


def kernel(x, index, dim_size):
    E, D = x.shape
    N = int(dim_size) if isinstance(dim_size, int) else 10000

    NC, NS = 2, 16
    NW = NC * NS
    EP = E // NW
    C = 128
    NB = 3
    NCH = EP // C
    CT = EP - NCH * C
    ROUNDS = (NCH + NB - 1) // NB
    assert ROUNDS * NB == NCH
    NPAD = ((N + NS * 8 - 1) // (NS * 8)) * (NS * 8)
    RPT = NPAD // NS

    idx2 = index.astype(jnp.int32)

    mesh = plsc.VectorSubcoreMesh(
        core_axis_name="c", subcore_axis_name="s", num_cores=NC, num_subcores=NS
    )

    @functools.partial(
        pl.kernel,
        mesh=mesh,
        out_type=jax.ShapeDtypeStruct((NC, NPAD, D), jnp.float32),
        scratch_types=[
            pltpu.VMEM((C, D), jnp.float32),
            pltpu.VMEM((C, D), jnp.float32),
            pltpu.VMEM((C, D), jnp.float32),
            pltpu.VMEM((C,), jnp.int32),
            pltpu.VMEM((C,), jnp.int32),
            pltpu.VMEM((C,), jnp.int32),
            pltpu.VMEM((CT,), jnp.int32),
            pltpu.VMEM_SHARED((NPAD, D), jnp.float32),
            pltpu.SemaphoreType.DMA,
            pltpu.SemaphoreType.DMA,
            pltpu.SemaphoreType.DMA,
            pltpu.SemaphoreType.DMA,
            pltpu.SemaphoreType.DMA,
            pltpu.SemaphoreType.DMA,
        ],
    )
    def seg_scatter(
        x_hbm, idx_hbm, part_hbm,
        xbuf0, xbuf1, xbuf2, ibuf0, ibuf1, ibuf2, ibuft,
        acc, lsem0, lsem1, lsem2, ssem0, ssem1, ssem2,
    ):
        c = lax.axis_index("c")
        s = lax.axis_index("s")
        w = c * NS + s
        base = w * EP

        xb = (xbuf0, xbuf1, xbuf2)
        ib = (ibuf0, ibuf1, ibuf2)
        ls = (lsem0, lsem1, lsem2)
        ss = (ssem0, ssem1, ssem2)

        def start_load(g, b):
            pltpu.async_copy(x_hbm.at[pl.ds(base + g * C, C)], xb[b], ls[b])
            pltpu.async_copy(idx_hbm.at[pl.ds(base + g * C, C)], ib[b], ls[b])

        def wait_load(b):
            pltpu.make_async_copy(x_hbm.at[pl.ds(0, C)], xb[b], ls[b]).wait()
            pltpu.make_async_copy(idx_hbm.at[pl.ds(0, C)], ib[b], ls[b]).wait()

        def start_scatter(b):
            pltpu.async_copy(xb[b], acc.at[ib[b]], ss[b], add=True)

        def wait_scatter(b):
            pltpu.make_async_copy(xb[b], acc.at[ib[b]], ss[b]).wait()

        zrow = jnp.zeros((16,), jnp.float32)

        def zstore(r, carry):
            for cg in range(D // 16):
                xbuf0[r, pl.ds(cg * 16, 16)] = zrow
            return carry

        lax.fori_loop(0, C, zstore, 0, unroll=2)
        for r0 in range(0, RPT, C):
            rl = min(C, RPT - r0)
            pltpu.sync_copy(
                xbuf0.at[pl.ds(0, rl)], acc.at[pl.ds(s * RPT + r0, rl)]
            )
        for b in range(NB):
            start_load(b, b)
        plsc.subcore_barrier()

        def round_body(i, carry):
            for b in range(NB):
                g = i * NB + b
                wait_load(b)
                start_scatter(b)
                pb = (b - 1) % NB
                pg = g - 1 + NB

                @pl.when(jnp.logical_and(g >= 1, pg < NCH))
                def _():
                    wait_scatter(pb)
                    start_load(pg, pb)

            return carry

        lax.fori_loop(0, ROUNDS, round_body, 0)

        for b in range(NB):
            wait_scatter(b)
        pltpu.sync_copy(x_hbm.at[pl.ds(base + NCH * C, CT)], xbuf0.at[pl.ds(0, CT)])
        pltpu.sync_copy(idx_hbm.at[pl.ds(base + NCH * C, CT)], ibuft)
        pltpu.sync_copy(xbuf0.at[pl.ds(0, CT)], acc.at[ibuft], add=True)

        plsc.subcore_barrier()
        pltpu.sync_copy(
            acc.at[pl.ds(s * RPT, RPT)],
            part_hbm.at[c].at[pl.ds(s * RPT, RPT)],
        )

    parts = seg_scatter(x, idx2)

    RB = NPAD // 8

    def merge_body(p_ref, o_ref):
        o_ref[...] = p_ref[0] + p_ref[1]

    out = pl.pallas_call(
        merge_body,
        grid=(NPAD // RB,),
        in_specs=[pl.BlockSpec((NC, RB, D), lambda i: (0, i, 0))],
        out_specs=pl.BlockSpec((RB, D), lambda i: (i, 0)),
        out_shape=jax.ShapeDtypeStruct((NPAD, D), jnp.float32),
    )(parts)
    return out[:N]


# speedup vs baseline: 1.1260x; 1.1260x over previous
import functools

import jax
import jax.numpy as jnp
from jax import lax
from jax.experimental import pallas as pl
from jax.experimental.pallas import tpu as pltpu
from jax.experimental.pallas import tpu_sc as plsc


def kernel(x, index, dim_size):
    E, D = x.shape
    N = int(dim_size) if isinstance(dim_size, int) else 10000

    NC, NS = 2, 16
    NW = NC * NS
    EP = E // NW
    C = 128
    NB = 3
    NCH = EP // C
    CT = EP - NCH * C
    ROUNDS = (NCH + NB - 1) // NB
    assert ROUNDS * NB == NCH
    NPAD = ((N + NS * 8 - 1) // (NS * 8)) * (NS * 8)
    RPT = NPAD // NS

    idx2 = index.astype(jnp.int32)

    mesh = plsc.VectorSubcoreMesh(
        core_axis_name="c", subcore_axis_name="s", num_cores=NC, num_subcores=NS
    )

    @functools.partial(
        pl.kernel,
        mesh=mesh,
        out_type=jax.ShapeDtypeStruct((NC, NPAD, D), jnp.float32),
        scratch_types=[
            pltpu.VMEM((C, D), jnp.float32),
            pltpu.VMEM((C, D), jnp.float32),
            pltpu.VMEM((C, D), jnp.float32),
            pltpu.VMEM((C,), jnp.int32),
            pltpu.VMEM((C,), jnp.int32),
            pltpu.VMEM((C,), jnp.int32),
            pltpu.VMEM((CT,), jnp.int32),
            pltpu.VMEM_SHARED((NPAD, D), jnp.float32),
            pltpu.SemaphoreType.DMA,
            pltpu.SemaphoreType.DMA,
            pltpu.SemaphoreType.DMA,
            pltpu.SemaphoreType.DMA,
            pltpu.SemaphoreType.DMA,
            pltpu.SemaphoreType.DMA,
        ],
    )
    def seg_scatter(
        x_hbm, idx_hbm, part_hbm,
        xbuf0, xbuf1, xbuf2, ibuf0, ibuf1, ibuf2, ibuft,
        acc, lsem0, lsem1, lsem2, ssem0, ssem1, ssem2,
    ):
        c = lax.axis_index("c")
        s = lax.axis_index("s")
        w = c * NS + s
        base = w * EP

        xb = (xbuf0, xbuf1, xbuf2)
        ib = (ibuf0, ibuf1, ibuf2)
        ls = (lsem0, lsem1, lsem2)
        ss = (ssem0, ssem1, ssem2)

        def start_load(g, b):
            pltpu.async_copy(x_hbm.at[pl.ds(base + g * C, C)], xb[b], ls[b])
            pltpu.async_copy(idx_hbm.at[pl.ds(base + g * C, C)], ib[b], ls[b])

        def wait_load(b):
            pltpu.make_async_copy(x_hbm.at[pl.ds(0, C)], xb[b], ls[b]).wait()
            pltpu.make_async_copy(idx_hbm.at[pl.ds(0, C)], ib[b], ls[b]).wait()

        def start_scatter(b):
            pltpu.async_copy(xb[b], acc.at[ib[b]], ss[b], add=True)

        def wait_scatter(b):
            pltpu.make_async_copy(xb[b], acc.at[ib[b]], ss[b]).wait()

        zrow = jnp.zeros((16,), jnp.float32)

        start_load(0, 0)
        start_load(1, 1)

        def zstore(r, carry):
            for cg in range(D // 16):
                xbuf2[r, pl.ds(cg * 16, 16)] = zrow
            return carry

        lax.fori_loop(0, C, zstore, 0, unroll=2)
        for r0 in range(0, RPT, C):
            rl = min(C, RPT - r0)
            pltpu.sync_copy(
                xbuf2.at[pl.ds(0, rl)], acc.at[pl.ds(s * RPT + r0, rl)]
            )
        start_load(2, 2)
        plsc.subcore_barrier()

        def round_body(i, carry):
            for b in range(NB):
                g = i * NB + b
                wait_load(b)
                start_scatter(b)

                @pl.when(g + NB < NCH)
                def _():
                    wait_scatter(b)
                    start_load(g + NB, b)

            return carry

        lax.fori_loop(0, ROUNDS, round_body, 0)

        for b in range(NB):
            wait_scatter(b)
        pltpu.sync_copy(x_hbm.at[pl.ds(base + NCH * C, CT)], xbuf0.at[pl.ds(0, CT)])
        pltpu.sync_copy(idx_hbm.at[pl.ds(base + NCH * C, CT)], ibuft)
        pltpu.sync_copy(xbuf0.at[pl.ds(0, CT)], acc.at[ibuft], add=True)

        plsc.subcore_barrier()
        pltpu.sync_copy(
            acc.at[pl.ds(s * RPT, RPT)],
            part_hbm.at[c].at[pl.ds(s * RPT, RPT)],
        )

    parts = seg_scatter(x, idx2)

    RB = 1000

    def merge_body(p_ref, o_ref):
        o_ref[...] = p_ref[0] + p_ref[1]

    out = pl.pallas_call(
        merge_body,
        grid=(N // RB,),
        in_specs=[pl.BlockSpec((NC, RB, D), lambda i: (0, i, 0))],
        out_specs=pl.BlockSpec((RB, D), lambda i: (i, 0)),
        out_shape=jax.ShapeDtypeStruct((N, D), jnp.float32),
    )(parts)
    return out


# speedup vs baseline: 1.1420x; 1.0142x over previous
import functools

import jax
import jax.numpy as jnp
from jax import lax
from jax.experimental import pallas as pl
from jax.experimental.pallas import tpu as pltpu
from jax.experimental.pallas import tpu_sc as plsc


def kernel(x, index, dim_size):
    E, D = x.shape
    N = int(dim_size) if isinstance(dim_size, int) else 10000

    NC, NS = 2, 16
    NW = NC * NS
    EP = E // NW
    C = 80
    NB = 4
    NCH = EP // C
    ROUNDS = (NCH - 1) // NB
    assert ROUNDS * NB + 1 == NCH
    NPAD = ((N + NS * 8 - 1) // (NS * 8)) * (NS * 8)
    RPT = NPAD // NS

    idx2 = index.astype(jnp.int32)

    mesh = plsc.VectorSubcoreMesh(
        core_axis_name="c", subcore_axis_name="s", num_cores=NC, num_subcores=NS
    )

    @functools.partial(
        pl.kernel,
        mesh=mesh,
        out_type=jax.ShapeDtypeStruct((NC, NPAD, D), jnp.float32),
        scratch_types=[
            pltpu.VMEM((C, D), jnp.float32),
            pltpu.VMEM((C, D), jnp.float32),
            pltpu.VMEM((C, D), jnp.float32),
            pltpu.VMEM((C, D), jnp.float32),
            pltpu.VMEM((C,), jnp.int32),
            pltpu.VMEM((C,), jnp.int32),
            pltpu.VMEM((C,), jnp.int32),
            pltpu.VMEM((C,), jnp.int32),
            pltpu.VMEM_SHARED((NPAD, D), jnp.float32),
            pltpu.SemaphoreType.DMA,
            pltpu.SemaphoreType.DMA,
            pltpu.SemaphoreType.DMA,
            pltpu.SemaphoreType.DMA,
            pltpu.SemaphoreType.DMA,
            pltpu.SemaphoreType.DMA,
            pltpu.SemaphoreType.DMA,
            pltpu.SemaphoreType.DMA,
        ],
    )
    def seg_scatter(
        x_hbm, idx_hbm, part_hbm,
        xbuf0, xbuf1, xbuf2, xbuf3, ibuf0, ibuf1, ibuf2, ibuf3,
        acc, lsem0, lsem1, lsem2, lsem3, ssem0, ssem1, ssem2, ssem3,
    ):
        c = lax.axis_index("c")
        s = lax.axis_index("s")
        w = c * NS + s
        base = w * EP

        xb = (xbuf0, xbuf1, xbuf2, xbuf3)
        ib = (ibuf0, ibuf1, ibuf2, ibuf3)
        ls = (lsem0, lsem1, lsem2, lsem3)
        ss = (ssem0, ssem1, ssem2, ssem3)

        def start_load(g, b):
            pltpu.async_copy(x_hbm.at[pl.ds(base + g * C, C)], xb[b], ls[b])
            pltpu.async_copy(idx_hbm.at[pl.ds(base + g * C, C)], ib[b], ls[b])

        def wait_load(b):
            pltpu.make_async_copy(x_hbm.at[pl.ds(0, C)], xb[b], ls[b]).wait()
            pltpu.make_async_copy(idx_hbm.at[pl.ds(0, C)], ib[b], ls[b]).wait()

        def start_scatter(b):
            pltpu.async_copy(xb[b], acc.at[ib[b]], ss[b], add=True)

        def wait_scatter(b):
            pltpu.make_async_copy(xb[b], acc.at[ib[b]], ss[b]).wait()

        zrow = jnp.zeros((16,), jnp.float32)

        start_load(0, 0)
        start_load(1, 1)
        start_load(2, 2)

        def zstore(r, carry):
            for cg in range(D // 16):
                xbuf3[r, pl.ds(cg * 16, 16)] = zrow
            return carry

        lax.fori_loop(0, C, zstore, 0, unroll=2)
        for r0 in range(0, RPT, C):
            rl = min(C, RPT - r0)
            pltpu.sync_copy(
                xbuf3.at[pl.ds(0, rl)], acc.at[pl.ds(s * RPT + r0, rl)]
            )
        plsc.subcore_barrier()

        def round_body(i, carry):
            for b in range(NB):
                g = i * NB + b
                pb = (b - 1) % NB

                @pl.when(g >= 1)
                def _():
                    wait_scatter(pb)

                @pl.when(g + NB - 1 < NCH)
                def _():
                    start_load(g + NB - 1, pb)

                wait_load(b)
                start_scatter(b)

            return carry

        lax.fori_loop(0, ROUNDS, round_body, 0)

        wait_scatter(NB - 1)
        wait_load(0)
        start_scatter(0)
        wait_scatter(0)

        plsc.subcore_barrier()
        pltpu.sync_copy(
            acc.at[pl.ds(s * RPT, RPT)],
            part_hbm.at[c].at[pl.ds(s * RPT, RPT)],
        )

    parts = seg_scatter(x, idx2)

    RB = 1000

    def merge_body(p_ref, o_ref):
        o_ref[...] = p_ref[0] + p_ref[1]

    out = pl.pallas_call(
        merge_body,
        grid=(N // RB,),
        in_specs=[pl.BlockSpec((NC, RB, D), lambda i: (0, i, 0))],
        out_specs=pl.BlockSpec((RB, D), lambda i: (i, 0)),
        out_shape=jax.ShapeDtypeStruct((N, D), jnp.float32),
    )(parts)
    return out
